# baseline (speedup 1.0000x reference)
import jax
import jax.numpy as jnp
from jax.experimental import pallas as pl
from jax.experimental.pallas import tpu as pltpu

N_MELS, R = 80, 5
PRENET_IN = N_MELS * R

F32 = jnp.float32


def _gru_update(gi, gh, h):
    H = h.shape[-1]
    r = jax.nn.sigmoid(gi[:, :H] + gh[:, :H])
    u = jax.nn.sigmoid(gi[:, H:2 * H] + gh[:, H:2 * H])
    n = jnp.tanh(gi[:, 2 * H:] + r * gh[:, 2 * H:])
    return (1.0 - u) * n + u * h


def _gru_merged(g, h):
    H = h.shape[-1]
    r = jax.nn.sigmoid(g[:, :H])
    u = jax.nn.sigmoid(g[:, H:2 * H])
    n = jnp.tanh(g[:, 2 * H:3 * H] + r * g[:, 3 * H:])
    return (1.0 - u) * n + u * h


def _dot(a, b):
    return jnp.dot(a, b, preferred_element_type=F32)


def _dotb(a, b_bf16):
    return jnp.dot(a.astype(jnp.bfloat16), b_bf16,
                   preferred_element_type=F32)


def _decoder_body(frames_ref, z_ref, maskf_ref,
                  preW1_ref, preb1_ref, preW2_ref, preb2_ref,
                  aWih_ref, abih_ref, aWhh_ref, abhh_ref,
                  Wq_ref, Wm_ref, v_ref,
                  pW_c_ref, pW_h_ref, pb_ref,
                  g1Wih_ref, g1bih_ref, g1Whh_ref, g1bhh_ref,
                  g2Wih_ref, g2bih_ref, g2Whh_ref, g2bhh_ref,
                  melW_ref, melb_ref,
                  mel_ref, al_ref,
                  pren_scr, zm_scr, xs_scr):
    T, Bc, _ = frames_ref.shape
    Tlen = z_ref.shape[1]

    fr = frames_ref[:].reshape(T * Bc, PRENET_IN)
    p1 = jnp.maximum(_dot(fr, preW1_ref[:]) + preb1_ref[:], 0.0)
    p2 = jnp.maximum(_dot(p1, preW2_ref[:]) + preb2_ref[:], 0.0)
    pren_scr[:] = p2.reshape(T, Bc, 128)

    zf = z_ref[:].reshape(Bc * Tlen, 256)
    zm_scr[:] = _dot(zf, Wm_ref[:]).astype(jnp.bfloat16).reshape(
        Bc, Tlen, 256)

    maskf = maskf_ref[:]
    v = v_ref[:]

    def step(t, carry):
        ha, h1, h2, ctx = carry
        pt = pren_scr[pl.ds(t, 1)].reshape(Bc, 128)
        gi = _dotb(jnp.concatenate([pt, ctx], -1), aWih_ref[:]) + abih_ref[:]
        gh = _dotb(ha, aWhh_ref[:]) + abhh_ref[:]
        ha = _gru_update(gi, gh, ha)
        q = _dotb(ha, Wq_ref[:]).astype(jnp.bfloat16)
        tt = jnp.tanh(zm_scr[:] + q[:, None, :])
        e = jnp.sum((tt * v[None, :, :].reshape(1, 1, 256)).astype(F32),
                    axis=-1)
        e = jnp.where(maskf > 0, e, -1e9)
        m = jnp.max(e, axis=-1, keepdims=True)
        ex = jnp.exp(e - m)
        a = ex / jnp.sum(ex, axis=-1, keepdims=True)
        ctx = jnp.sum(a[:, :, None] * z_ref[:], axis=1)
        x = _dotb(ctx, pW_c_ref[:]) + _dotb(ha, pW_h_ref[:]) + pb_ref[:]
        h1 = _gru_update(_dotb(x, g1Wih_ref[:]) + g1bih_ref[:],
                         _dotb(h1, g1Whh_ref[:]) + g1bhh_ref[:], h1)
        x = x + h1
        h2 = _gru_update(_dotb(x, g2Wih_ref[:]) + g2bih_ref[:],
                         _dotb(h2, g2Whh_ref[:]) + g2bhh_ref[:], h2)
        x = x + h2
        xs_scr[pl.ds(t, 1)] = x.reshape(1, Bc, 256)
        al_ref[pl.ds(t, 1)] = a.reshape(1, Bc, Tlen)
        return ha, h1, h2, ctx

    z0 = jnp.zeros((Bc, 256), F32)
    jax.lax.fori_loop(0, T, step, (z0, z0, z0, z0))

    xs = xs_scr[:].reshape(T * Bc, 256)
    mel_ref[:] = (_dot(xs, melW_ref[:]) + melb_ref[:]).reshape(
        T, Bc, PRENET_IN)


def _cbhg_body(mel_ref, preW_ref,
               bW1, bW2, bW3, bW4, bW5, bW6, bW7, bW8,
               bscale_ref, bshift_ref,
               p1W_ref, p1s_ref, p1sh_ref,
               p2W_ref, p2s_ref, p2sh_ref,
               hWh_ref, hbh_ref, hWt_ref, hbt_ref,
               out_ref,
               xp_scr, bank_scr, pp_scr, c1p_scr):
    L = mel_ref.shape[1]
    x = mel_ref[0]
    x0 = _dot(x, preW_ref[:])

    xp_scr[:] = jnp.concatenate(
        [jnp.zeros((8, 128), F32), x0, jnp.zeros((4, 128), F32)], axis=0)
    bank_scr[:] = jnp.full((512, 1024), -jnp.inf, F32)
    bank_refs = (bW1, bW2, bW3, bW4, bW5, bW6, bW7, bW8)
    for k in range(1, 9):
        Wk = bank_refs[k - 1]
        acc = jnp.zeros((L, 128), F32)
        for j in range(k):
            s = j - k // 2 + 8
            acc = acc + _dot(xp_scr[s:s + L], Wk[j])
        yk = jnp.maximum(acc * bscale_ref[k - 1] + bshift_ref[k - 1], 0.0)
        bank_scr[0:L, (k - 1) * 128:k * 128] = yk

    pooled = jnp.maximum(bank_scr[0:L], bank_scr[1:L + 1])

    pp_scr[:] = jnp.concatenate(
        [jnp.zeros((8, 1024), F32), pooled, jnp.zeros((4, 1024), F32)], axis=0)
    c1 = jnp.zeros((L, 256), F32)
    for j in range(3):
        c1 = c1 + _dot(pp_scr[7 + j:7 + j + L], p1W_ref[j])
    c1 = jnp.maximum(c1 * p1s_ref[:] + p1sh_ref[:], 0.0)

    c1p_scr[:] = jnp.concatenate(
        [jnp.zeros((8, 256), F32), c1, jnp.zeros((4, 256), F32)], axis=0)
    c2 = jnp.zeros((L, 128), F32)
    for j in range(3):
        c2 = c2 + _dot(c1p_scr[7 + j:7 + j + L], p2W_ref[j])
    xh = c2 * p2s_ref[:] + p2sh_ref[:] + x0

    for i in range(4):
        Hh = jnp.maximum(_dot(xh, hWh_ref[i]) + hbh_ref[i:i + 1], 0.0)
        Tt = jax.nn.sigmoid(_dot(xh, hWt_ref[i]) + hbt_ref[i:i + 1])
        xh = Hh * Tt + xh * (1.0 - Tt)
    out_ref[0] = xh


def _bigru_body(x_ref, Wih_ref, bih_ref, Whh_ref, bhh_ref, out_ref):
    Bsz = x_ref.shape[0]
    L = x_ref.shape[1] // 128
    UNROLL = 4
    pid = pl.program_id(0)
    Wih = Wih_ref[0]
    bih = bih_ref[0]
    Whh = Whh_ref[0]
    bhh = bhh_ref[0]

    def step(j, h):
        base = j * UNROLL
        idxs = [jnp.where(pid == 0, base + k, L - 1 - (base + k))
                for k in range(UNROLL)]
        gis = [_dot(x_ref[:, pl.ds(pl.multiple_of(idx * 128, 128), 128)],
                    Wih) + bih
               for idx in idxs]
        for k in range(UNROLL):
            gh = _dot(h, Whh) + bhh
            h = _gru_update(gis[k], gh, h)
            out_ref[0, pl.ds(idxs[k], 1)] = h.reshape(1, Bsz, 128)
        return h

    jax.lax.fori_loop(0, L // UNROLL, step, jnp.zeros((Bsz, 128), F32))


def _linear_body(xf_ref, xb_ref, Wf_ref, Wb_ref, b_ref, out_ref):
    out_ref[0] = (_dot(xf_ref[:], Wf_ref[:]) + _dot(xb_ref[:], Wb_ref[:])
                  + b_ref[:])


def _full_spec(shape):
    n = len(shape)
    return pl.BlockSpec(shape, lambda i, _n=n: (0,) * _n)


def kernel(z, y, lengths, params):
    p = params
    Bsz, Tlen, _ = z.shape
    T = y.shape[1] // R
    L = T * R
    Bc = Bsz // 2

    yr = y.reshape(Bsz, T, PRENET_IN)
    frames = jnp.concatenate(
        [jnp.zeros((Bsz, 1, PRENET_IN), z.dtype), yr[:, :-1]], axis=1)
    frames_t = jnp.swapaxes(frames, 0, 1)
    maskf = (jnp.arange(Tlen)[None, :] < lengths[:, None]).astype(F32)

    row = lambda b: b[None, :]

    def merge_gru(g):
        Wih, Whh, bih, bhh = g['Wih'], g['Whh'], g['bih'], g['bhh']
        H = Whh.shape[0]
        i = Wih.shape[0]
        top = jnp.concatenate(
            [Wih[:, :2 * H], Wih[:, 2 * H:], jnp.zeros((i, H), F32)], axis=1)
        bot = jnp.concatenate(
            [Whh[:, :2 * H], jnp.zeros((H, H), F32), Whh[:, 2 * H:]], axis=1)
        W = jnp.concatenate([top, bot], axis=0)
        b = jnp.concatenate(
            [bih[:2 * H] + bhh[:2 * H], bih[2 * H:], bhh[2 * H:]])
        return W, row(b)

    arnn, g1, g2 = p['attn_rnn'], p['dec_gru1'], p['dec_gru2']
    bf = lambda w: w.astype(jnp.bfloat16)
    dec_weights = (
        p['pre_W1'], row(p['pre_b1']), p['pre_W2'], row(p['pre_b2']),
        bf(arnn['Wih']), row(arnn['bih']), bf(arnn['Whh']), row(arnn['bhh']),
        bf(p['Wq']), p['Wm'], row(p['v']).astype(jnp.bfloat16),
        bf(p['proj_W'][:256]), bf(p['proj_W'][256:]), row(p['proj_b']),
        bf(g1['Wih']), row(g1['bih']), bf(g1['Whh']), row(g1['bhh']),
        bf(g2['Wih']), row(g2['bih']), bf(g2['Whh']), row(g2['bhh']),
        p['mel_W'], row(p['mel_b']),
    )
    dec_in_specs = (
        [pl.BlockSpec((T, Bc, PRENET_IN), lambda i: (0, i, 0)),
         pl.BlockSpec((Bc, Tlen, 256), lambda i: (i, 0, 0)),
         pl.BlockSpec((Bc, Tlen), lambda i: (i, 0))]
        + [_full_spec(w.shape) for w in dec_weights])
    mels, aligns = pl.pallas_call(
        _decoder_body,
        grid=(2,),
        in_specs=dec_in_specs,
        out_specs=[pl.BlockSpec((T, Bc, PRENET_IN), lambda i: (0, i, 0)),
                   pl.BlockSpec((T, Bc, Tlen), lambda i: (0, i, 0))],
        out_shape=[jax.ShapeDtypeStruct((T, Bsz, PRENET_IN), F32),
                   jax.ShapeDtypeStruct((T, Bsz, Tlen), F32)],
        scratch_shapes=[pltpu.VMEM((T, Bc, 128), F32),
                        pltpu.VMEM((Bc, Tlen, 256), jnp.bfloat16),
                        pltpu.VMEM((T, Bc, 256), F32)],
        compiler_params=pltpu.CompilerParams(
            dimension_semantics=("parallel",),
            vmem_limit_bytes=56 * 1024 * 1024),
    )(frames_t, z, maskf, *dec_weights)

    mel_pred = jnp.swapaxes(mels, 0, 1).reshape(Bsz, L, N_MELS)
    alignments = jnp.swapaxes(aligns, 0, 1)

    def bn_scale_shift(bn):
        s = bn['gamma'] * jax.lax.rsqrt(bn['var'] + 1e-5)
        return s, bn['beta'] - bn['mean'] * s

    bss = [bn_scale_shift(bp['bn']) for bp in p['bank']]
    bscale = jnp.stack([s for s, _ in bss])
    bshift = jnp.stack([sh for _, sh in bss])
    p1s, p1sh = bn_scale_shift(p['proj1_bn'])
    p2s, p2sh = bn_scale_shift(p['proj2_bn'])
    hWh = jnp.stack([hp['Wh'] for hp in p['highway']])
    hbh = jnp.stack([hp['bh'] for hp in p['highway']])
    hWt = jnp.stack([hp['Wt'] for hp in p['highway']])
    hbt = jnp.stack([hp['bt'] for hp in p['highway']])
    cbhg_weights = (
        (p['pre_cbhg_W'],)
        + tuple(bp['W'] for bp in p['bank'])
        + (bscale, bshift,
           p['proj1_W'], row(p1s), row(p1sh),
           p['proj2_W'], row(p2s), row(p2sh),
           hWh, hbh, hWt, hbt))
    xcb = pl.pallas_call(
        _cbhg_body,
        grid=(Bsz,),
        in_specs=([pl.BlockSpec((1, L, N_MELS), lambda b: (b, 0, 0))]
                  + [_full_spec(w.shape) for w in cbhg_weights]),
        out_specs=pl.BlockSpec((1, L, 128), lambda b: (b, 0, 0)),
        out_shape=jax.ShapeDtypeStruct((Bsz, L, 128), F32),
        scratch_shapes=[pltpu.VMEM((512, 128), F32),
                        pltpu.VMEM((512, 1024), F32),
                        pltpu.VMEM((512, 1024), F32),
                        pltpu.VMEM((512, 256), F32)],
        compiler_params=pltpu.CompilerParams(
            dimension_semantics=("parallel",),
            vmem_limit_bytes=56 * 1024 * 1024),
    )(mel_pred, *cbhg_weights)

    xcb_flat = xcb.reshape(Bsz, L * 128)
    Wih_fb = jnp.stack([p['gru_f']['Wih'], p['gru_b']['Wih']])
    bih_fb = jnp.stack([row(p['gru_f']['bih']), row(p['gru_b']['bih'])])
    Whh_fb = jnp.stack([p['gru_f']['Whh'], p['gru_b']['Whh']])
    bhh_fb = jnp.stack([row(p['gru_f']['bhh']), row(p['gru_b']['bhh'])])
    h_all = pl.pallas_call(
        _bigru_body,
        grid=(2,),
        in_specs=[pl.BlockSpec((Bsz, L * 128), lambda i: (0, 0)),
                  pl.BlockSpec((1, 128, 384), lambda i: (i, 0, 0)),
                  pl.BlockSpec((1, 1, 384), lambda i: (i, 0, 0)),
                  pl.BlockSpec((1, 128, 384), lambda i: (i, 0, 0)),
                  pl.BlockSpec((1, 1, 384), lambda i: (i, 0, 0))],
        out_specs=pl.BlockSpec((1, L, Bsz, 128), lambda i: (i, 0, 0, 0)),
        out_shape=jax.ShapeDtypeStruct((2, L, Bsz, 128), F32),
        compiler_params=pltpu.CompilerParams(
            dimension_semantics=("parallel",),
            vmem_limit_bytes=56 * 1024 * 1024),
    )(xcb_flat, Wih_fb, bih_fb, Whh_fb, bhh_fb)

    xf = h_all[0].reshape(L, Bsz * 128)
    xb = h_all[1].reshape(L, Bsz * 128)
    lin_pred = pl.pallas_call(
        _linear_body,
        grid=(Bsz,),
        in_specs=[pl.BlockSpec((L, 128), lambda b: (0, b)),
                  pl.BlockSpec((L, 128), lambda b: (0, b)),
                  _full_spec((128, 1025)), _full_spec((128, 1025)),
                  _full_spec((1, 1025))],
        out_specs=pl.BlockSpec((1, L, 1025), lambda b: (b, 0, 0)),
        out_shape=jax.ShapeDtypeStruct((Bsz, L, 1025), F32),
        compiler_params=pltpu.CompilerParams(
            dimension_semantics=("parallel",),
            vmem_limit_bytes=56 * 1024 * 1024),
    )(xf, xb, p['lin_W'][:128], p['lin_W'][128:], row(p['lin_b']))

    return mel_pred, lin_pred, alignments


# speedup vs baseline: 1.0300x; 1.0300x over previous
import jax
import jax.numpy as jnp
from jax.experimental import pallas as pl
from jax.experimental.pallas import tpu as pltpu

N_MELS, R = 80, 5
PRENET_IN = N_MELS * R

F32 = jnp.float32


def _gru_update(gi, gh, h):
    H = h.shape[-1]
    r = jax.nn.sigmoid(gi[:, :H] + gh[:, :H])
    u = jax.nn.sigmoid(gi[:, H:2 * H] + gh[:, H:2 * H])
    n = jnp.tanh(gi[:, 2 * H:] + r * gh[:, 2 * H:])
    return (1.0 - u) * n + u * h


def _gru_merged(g, h):
    H = h.shape[-1]
    r = jax.nn.sigmoid(g[:, :H])
    u = jax.nn.sigmoid(g[:, H:2 * H])
    n = jnp.tanh(g[:, 2 * H:3 * H] + r * g[:, 3 * H:])
    return (1.0 - u) * n + u * h


def _dot(a, b):
    return jnp.dot(a, b, preferred_element_type=F32)


def _dotb(a, b_bf16):
    return jnp.dot(a.astype(jnp.bfloat16), b_bf16,
                   preferred_element_type=F32)


def _decoder_body(frames_ref, z_ref, maskf_ref,
                  preW1_ref, preb1_ref, preW2_ref, preb2_ref,
                  aWih_ref, abih_ref, aWhh_ref, abhh_ref,
                  Wq_ref, Wm_ref, v_ref,
                  pW_c_ref, pW_h_ref, pb_ref,
                  g1Wih_ref, g1bih_ref, g1Whh_ref, g1bhh_ref,
                  g2Wih_ref, g2bih_ref, g2Whh_ref, g2bhh_ref,
                  melW_ref, melb_ref,
                  mel_ref, al_ref,
                  pren_scr, zm_scr, xs_scr):
    T, Bc, _ = frames_ref.shape
    Tlen = z_ref.shape[1]

    fr = frames_ref[:].reshape(T * Bc, PRENET_IN)
    p1 = jnp.maximum(_dot(fr, preW1_ref[:]) + preb1_ref[:], 0.0)
    p2 = jnp.maximum(_dot(p1, preW2_ref[:]) + preb2_ref[:], 0.0)
    pren_scr[:] = p2.reshape(T, Bc, 128)

    zf = z_ref[:].reshape(Bc * Tlen, 256)
    zm_scr[:] = _dot(zf, Wm_ref[:]).astype(jnp.bfloat16).reshape(
        Bc, Tlen, 256)

    maskf = maskf_ref[:]
    v = v_ref[:]

    def step(t, carry):
        ha, h1, h2, ctx = carry
        pt = pren_scr[pl.ds(t, 1)].reshape(Bc, 128)
        gi = _dot(jnp.concatenate([pt, ctx], -1), aWih_ref[:]) + abih_ref[:]
        gh = _dot(ha, aWhh_ref[:]) + abhh_ref[:]
        ha = _gru_update(gi, gh, ha)
        q = _dot(ha, Wq_ref[:]).astype(jnp.bfloat16)
        tt = jnp.tanh(zm_scr[:] + q[:, None, :])
        e = jnp.sum((tt * v[None, :, :].reshape(1, 1, 256)).astype(F32),
                    axis=-1)
        e = jnp.where(maskf > 0, e, -1e9)
        m = jnp.max(e, axis=-1, keepdims=True)
        ex = jnp.exp(e - m)
        a = ex / jnp.sum(ex, axis=-1, keepdims=True)
        ctx = jnp.sum(a[:, :, None] * z_ref[:], axis=1)
        x = _dot(ctx, pW_c_ref[:]) + _dot(ha, pW_h_ref[:]) + pb_ref[:]
        h1 = _gru_update(_dot(x, g1Wih_ref[:]) + g1bih_ref[:],
                         _dot(h1, g1Whh_ref[:]) + g1bhh_ref[:], h1)
        x = x + h1
        h2 = _gru_update(_dot(x, g2Wih_ref[:]) + g2bih_ref[:],
                         _dot(h2, g2Whh_ref[:]) + g2bhh_ref[:], h2)
        x = x + h2
        xs_scr[pl.ds(t, 1)] = x.reshape(1, Bc, 256)
        al_ref[pl.ds(t, 1)] = a.reshape(1, Bc, Tlen)
        return ha, h1, h2, ctx

    z0 = jnp.zeros((Bc, 256), F32)
    jax.lax.fori_loop(0, T, step, (z0, z0, z0, z0))

    xs = xs_scr[:].reshape(T * Bc, 256)
    mel_ref[:] = (_dot(xs, melW_ref[:]) + melb_ref[:]).reshape(
        T, Bc, PRENET_IN)


def _cbhg_body(mel_ref, preW_ref,
               bW1, bW2, bW3, bW4, bW5, bW6, bW7, bW8,
               bscale_ref, bshift_ref,
               p1W_ref, p1s_ref, p1sh_ref,
               p2W_ref, p2s_ref, p2sh_ref,
               hWh_ref, hbh_ref, hWt_ref, hbt_ref,
               out_ref,
               xp_scr, bank_scr, pp_scr, c1p_scr):
    L = mel_ref.shape[1]
    x = mel_ref[0]
    x0 = _dot(x, preW_ref[:])

    xp_scr[:] = jnp.concatenate(
        [jnp.zeros((8, 128), F32), x0, jnp.zeros((4, 128), F32)], axis=0)
    bank_scr[:] = jnp.full((512, 1024), -jnp.inf, F32)
    bank_refs = (bW1, bW2, bW3, bW4, bW5, bW6, bW7, bW8)
    for k in range(1, 9):
        Wk = bank_refs[k - 1]
        acc = jnp.zeros((L, 128), F32)
        for j in range(k):
            s = j - k // 2 + 8
            acc = acc + _dot(xp_scr[s:s + L], Wk[j])
        yk = jnp.maximum(acc * bscale_ref[k - 1] + bshift_ref[k - 1], 0.0)
        bank_scr[0:L, (k - 1) * 128:k * 128] = yk

    pooled = jnp.maximum(bank_scr[0:L], bank_scr[1:L + 1])

    pp_scr[:] = jnp.concatenate(
        [jnp.zeros((8, 1024), F32), pooled, jnp.zeros((4, 1024), F32)], axis=0)
    c1 = jnp.zeros((L, 256), F32)
    for j in range(3):
        c1 = c1 + _dot(pp_scr[7 + j:7 + j + L], p1W_ref[j])
    c1 = jnp.maximum(c1 * p1s_ref[:] + p1sh_ref[:], 0.0)

    c1p_scr[:] = jnp.concatenate(
        [jnp.zeros((8, 256), F32), c1, jnp.zeros((4, 256), F32)], axis=0)
    c2 = jnp.zeros((L, 128), F32)
    for j in range(3):
        c2 = c2 + _dot(c1p_scr[7 + j:7 + j + L], p2W_ref[j])
    xh = c2 * p2s_ref[:] + p2sh_ref[:] + x0

    for i in range(4):
        Hh = jnp.maximum(_dot(xh, hWh_ref[i]) + hbh_ref[i:i + 1], 0.0)
        Tt = jax.nn.sigmoid(_dot(xh, hWt_ref[i]) + hbt_ref[i:i + 1])
        xh = Hh * Tt + xh * (1.0 - Tt)
    out_ref[0] = xh


def _bigru_body(x_ref, Wih_ref, bih_ref, Whh_ref, bhh_ref, out_ref):
    Bsz = x_ref.shape[0]
    L = x_ref.shape[1] // 128
    UNROLL = 10
    pid = pl.program_id(0)
    Wih = Wih_ref[0]
    bih = bih_ref[0]
    Whh = Whh_ref[0]
    bhh = bhh_ref[0]

    def step(j, h):
        base = j * UNROLL
        idxs = [jnp.where(pid == 0, base + k, L - 1 - (base + k))
                for k in range(UNROLL)]
        gis = [_dot(x_ref[:, pl.ds(pl.multiple_of(idx * 128, 128), 128)],
                    Wih) + bih
               for idx in idxs]
        for k in range(UNROLL):
            gh = _dot(h, Whh) + bhh
            h = _gru_update(gis[k], gh, h)
            out_ref[0, pl.ds(idxs[k], 1)] = h.reshape(1, Bsz, 128)
        return h

    jax.lax.fori_loop(0, L // UNROLL, step, jnp.zeros((Bsz, 128), F32))


def _linear_body(xf_ref, xb_ref, Wf_ref, Wb_ref, b_ref, out_ref):
    out_ref[0] = (_dot(xf_ref[:], Wf_ref[:]) + _dot(xb_ref[:], Wb_ref[:])
                  + b_ref[:])


def _full_spec(shape):
    n = len(shape)
    return pl.BlockSpec(shape, lambda i, _n=n: (0,) * _n)


def kernel(z, y, lengths, params):
    p = params
    Bsz, Tlen, _ = z.shape
    T = y.shape[1] // R
    L = T * R
    Bc = Bsz // 2

    yr = y.reshape(Bsz, T, PRENET_IN)
    frames = jnp.concatenate(
        [jnp.zeros((Bsz, 1, PRENET_IN), z.dtype), yr[:, :-1]], axis=1)
    frames_t = jnp.swapaxes(frames, 0, 1)
    maskf = (jnp.arange(Tlen)[None, :] < lengths[:, None]).astype(F32)

    row = lambda b: b[None, :]

    def merge_gru(g):
        Wih, Whh, bih, bhh = g['Wih'], g['Whh'], g['bih'], g['bhh']
        H = Whh.shape[0]
        i = Wih.shape[0]
        top = jnp.concatenate(
            [Wih[:, :2 * H], Wih[:, 2 * H:], jnp.zeros((i, H), F32)], axis=1)
        bot = jnp.concatenate(
            [Whh[:, :2 * H], jnp.zeros((H, H), F32), Whh[:, 2 * H:]], axis=1)
        W = jnp.concatenate([top, bot], axis=0)
        b = jnp.concatenate(
            [bih[:2 * H] + bhh[:2 * H], bih[2 * H:], bhh[2 * H:]])
        return W, row(b)

    arnn, g1, g2 = p['attn_rnn'], p['dec_gru1'], p['dec_gru2']
    dec_weights = (
        p['pre_W1'], row(p['pre_b1']), p['pre_W2'], row(p['pre_b2']),
        arnn['Wih'], row(arnn['bih']), arnn['Whh'], row(arnn['bhh']),
        p['Wq'], p['Wm'], row(p['v']).astype(jnp.bfloat16),
        p['proj_W'][:256], p['proj_W'][256:], row(p['proj_b']),
        g1['Wih'], row(g1['bih']), g1['Whh'], row(g1['bhh']),
        g2['Wih'], row(g2['bih']), g2['Whh'], row(g2['bhh']),
        p['mel_W'], row(p['mel_b']),
    )
    dec_in_specs = (
        [pl.BlockSpec((T, Bc, PRENET_IN), lambda i: (0, i, 0)),
         pl.BlockSpec((Bc, Tlen, 256), lambda i: (i, 0, 0)),
         pl.BlockSpec((Bc, Tlen), lambda i: (i, 0))]
        + [_full_spec(w.shape) for w in dec_weights])
    mels, aligns = pl.pallas_call(
        _decoder_body,
        grid=(2,),
        in_specs=dec_in_specs,
        out_specs=[pl.BlockSpec((T, Bc, PRENET_IN), lambda i: (0, i, 0)),
                   pl.BlockSpec((T, Bc, Tlen), lambda i: (0, i, 0))],
        out_shape=[jax.ShapeDtypeStruct((T, Bsz, PRENET_IN), F32),
                   jax.ShapeDtypeStruct((T, Bsz, Tlen), F32)],
        scratch_shapes=[pltpu.VMEM((T, Bc, 128), F32),
                        pltpu.VMEM((Bc, Tlen, 256), jnp.bfloat16),
                        pltpu.VMEM((T, Bc, 256), F32)],
        compiler_params=pltpu.CompilerParams(
            dimension_semantics=("parallel",),
            vmem_limit_bytes=56 * 1024 * 1024),
    )(frames_t, z, maskf, *dec_weights)

    mel_pred = jnp.swapaxes(mels, 0, 1).reshape(Bsz, L, N_MELS)
    alignments = jnp.swapaxes(aligns, 0, 1)

    def bn_scale_shift(bn):
        s = bn['gamma'] * jax.lax.rsqrt(bn['var'] + 1e-5)
        return s, bn['beta'] - bn['mean'] * s

    bss = [bn_scale_shift(bp['bn']) for bp in p['bank']]
    bscale = jnp.stack([s for s, _ in bss])
    bshift = jnp.stack([sh for _, sh in bss])
    p1s, p1sh = bn_scale_shift(p['proj1_bn'])
    p2s, p2sh = bn_scale_shift(p['proj2_bn'])
    hWh = jnp.stack([hp['Wh'] for hp in p['highway']])
    hbh = jnp.stack([hp['bh'] for hp in p['highway']])
    hWt = jnp.stack([hp['Wt'] for hp in p['highway']])
    hbt = jnp.stack([hp['bt'] for hp in p['highway']])
    cbhg_weights = (
        (p['pre_cbhg_W'],)
        + tuple(bp['W'] for bp in p['bank'])
        + (bscale, bshift,
           p['proj1_W'], row(p1s), row(p1sh),
           p['proj2_W'], row(p2s), row(p2sh),
           hWh, hbh, hWt, hbt))
    xcb = pl.pallas_call(
        _cbhg_body,
        grid=(Bsz,),
        in_specs=([pl.BlockSpec((1, L, N_MELS), lambda b: (b, 0, 0))]
                  + [_full_spec(w.shape) for w in cbhg_weights]),
        out_specs=pl.BlockSpec((1, L, 128), lambda b: (b, 0, 0)),
        out_shape=jax.ShapeDtypeStruct((Bsz, L, 128), F32),
        scratch_shapes=[pltpu.VMEM((512, 128), F32),
                        pltpu.VMEM((512, 1024), F32),
                        pltpu.VMEM((512, 1024), F32),
                        pltpu.VMEM((512, 256), F32)],
        compiler_params=pltpu.CompilerParams(
            dimension_semantics=("parallel",),
            vmem_limit_bytes=56 * 1024 * 1024),
    )(mel_pred, *cbhg_weights)

    xcb_flat = xcb.reshape(Bsz, L * 128)
    Wih_fb = jnp.stack([p['gru_f']['Wih'], p['gru_b']['Wih']])
    bih_fb = jnp.stack([row(p['gru_f']['bih']), row(p['gru_b']['bih'])])
    Whh_fb = jnp.stack([p['gru_f']['Whh'], p['gru_b']['Whh']])
    bhh_fb = jnp.stack([row(p['gru_f']['bhh']), row(p['gru_b']['bhh'])])
    h_all = pl.pallas_call(
        _bigru_body,
        grid=(2,),
        in_specs=[pl.BlockSpec((Bsz, L * 128), lambda i: (0, 0)),
                  pl.BlockSpec((1, 128, 384), lambda i: (i, 0, 0)),
                  pl.BlockSpec((1, 1, 384), lambda i: (i, 0, 0)),
                  pl.BlockSpec((1, 128, 384), lambda i: (i, 0, 0)),
                  pl.BlockSpec((1, 1, 384), lambda i: (i, 0, 0))],
        out_specs=pl.BlockSpec((1, L, Bsz, 128), lambda i: (i, 0, 0, 0)),
        out_shape=jax.ShapeDtypeStruct((2, L, Bsz, 128), F32),
        compiler_params=pltpu.CompilerParams(
            dimension_semantics=("parallel",),
            vmem_limit_bytes=56 * 1024 * 1024),
    )(xcb_flat, Wih_fb, bih_fb, Whh_fb, bhh_fb)

    xf = h_all[0].reshape(L, Bsz * 128)
    xb = h_all[1].reshape(L, Bsz * 128)
    lin_pred = pl.pallas_call(
        _linear_body,
        grid=(Bsz,),
        in_specs=[pl.BlockSpec((L, 128), lambda b: (0, b)),
                  pl.BlockSpec((L, 128), lambda b: (0, b)),
                  _full_spec((128, 1025)), _full_spec((128, 1025)),
                  _full_spec((1, 1025))],
        out_specs=pl.BlockSpec((1, L, 1025), lambda b: (b, 0, 0)),
        out_shape=jax.ShapeDtypeStruct((Bsz, L, 1025), F32),
        compiler_params=pltpu.CompilerParams(
            dimension_semantics=("parallel",),
            vmem_limit_bytes=56 * 1024 * 1024),
    )(xf, xb, p['lin_W'][:128], p['lin_W'][128:], row(p['lin_b']))

    return mel_pred, lin_pred, alignments


# speedup vs baseline: 1.0315x; 1.0015x over previous
import jax
import jax.numpy as jnp
from jax.experimental import pallas as pl
from jax.experimental.pallas import tpu as pltpu

N_MELS, R = 80, 5
PRENET_IN = N_MELS * R

F32 = jnp.float32


def _gru_update(gi, gh, h):
    H = h.shape[-1]
    r = jax.nn.sigmoid(gi[:, :H] + gh[:, :H])
    u = jax.nn.sigmoid(gi[:, H:2 * H] + gh[:, H:2 * H])
    n = jnp.tanh(gi[:, 2 * H:] + r * gh[:, 2 * H:])
    return (1.0 - u) * n + u * h


def _dot(a, b):
    return jnp.dot(a, b, preferred_element_type=F32)


def _decoder_body(frames_ref, z_ref, maskf_ref,
                  preW1_ref, preb1_ref, preW2_ref, preb2_ref,
                  aWih_ref, abih_ref, aWhh_ref, abhh_ref,
                  Wq_ref, Wm_ref, v_ref,
                  pW_c_ref, pW_h_ref, pb_ref,
                  g1Wih_ref, g1bih_ref, g1Whh_ref, g1bhh_ref,
                  g2Wih_ref, g2bih_ref, g2Whh_ref, g2bhh_ref,
                  melW_ref, melb_ref,
                  mel_ref, al_ref,
                  pren_scr, zm_scr, xs_scr):
    T, Bc, _ = frames_ref.shape
    Tlen = z_ref.shape[1]

    fr = frames_ref[:].reshape(T * Bc, PRENET_IN)
    p1 = jnp.maximum(_dot(fr, preW1_ref[:]) + preb1_ref[:], 0.0)
    p2 = jnp.maximum(_dot(p1, preW2_ref[:]) + preb2_ref[:], 0.0)
    pren_scr[:] = p2.reshape(T, Bc, 128)

    zf = z_ref[:].reshape(Bc * Tlen, 256)
    zm_scr[:] = _dot(zf, Wm_ref[:]).astype(jnp.bfloat16).reshape(
        Bc, Tlen, 256)

    maskf = maskf_ref[:]
    v = v_ref[:]

    def step(t, carry):
        ha, h1, h2, ctx = carry
        pt = pren_scr[pl.ds(t, 1)].reshape(Bc, 128)
        gi = _dot(jnp.concatenate([pt, ctx], -1), aWih_ref[:]) + abih_ref[:]
        gh = _dot(ha, aWhh_ref[:]) + abhh_ref[:]
        ha = _gru_update(gi, gh, ha)
        q = _dot(ha, Wq_ref[:]).astype(jnp.bfloat16)
        tt = jnp.tanh(zm_scr[:] + q[:, None, :])
        e = jnp.sum((tt * v[None, :, :].reshape(1, 1, 256)).astype(F32),
                    axis=-1)
        e = jnp.where(maskf > 0, e, -1e9)
        m = jnp.max(e, axis=-1, keepdims=True)
        ex = jnp.exp(e - m)
        a = ex / jnp.sum(ex, axis=-1, keepdims=True)
        ctx = jnp.sum(a[:, :, None] * z_ref[:], axis=1)
        x = _dot(ctx, pW_c_ref[:]) + _dot(ha, pW_h_ref[:]) + pb_ref[:]
        h1 = _gru_update(_dot(x, g1Wih_ref[:]) + g1bih_ref[:],
                         _dot(h1, g1Whh_ref[:]) + g1bhh_ref[:], h1)
        x = x + h1
        h2 = _gru_update(_dot(x, g2Wih_ref[:]) + g2bih_ref[:],
                         _dot(h2, g2Whh_ref[:]) + g2bhh_ref[:], h2)
        x = x + h2
        xs_scr[pl.ds(t, 1)] = x.reshape(1, Bc, 256)
        al_ref[pl.ds(t, 1)] = a.reshape(1, Bc, Tlen)
        return ha, h1, h2, ctx

    z0 = jnp.zeros((Bc, 256), F32)
    jax.lax.fori_loop(0, T, step, (z0, z0, z0, z0))

    xs = xs_scr[:].reshape(T * Bc, 256)
    mel_ref[:] = (_dot(xs, melW_ref[:]) + melb_ref[:]).reshape(
        T, Bc, PRENET_IN)


def _cbhg_body(mel_ref, preW_ref,
               bW1, bW2, bW3, bW4, bW5, bW6, bW7, bW8,
               bscale_ref, bshift_ref,
               p1W_ref, p1s_ref, p1sh_ref,
               p2W_ref, p2s_ref, p2sh_ref,
               hWh_ref, hbh_ref, hWt_ref, hbt_ref,
               out_ref,
               xp_scr, bank_scr, pp_scr, c1p_scr):
    L = mel_ref.shape[1]
    x = mel_ref[0]
    x0 = _dot(x, preW_ref[:])

    xp_scr[:] = jnp.concatenate(
        [jnp.zeros((8, 128), F32), x0, jnp.zeros((4, 128), F32)], axis=0)
    bank_scr[:] = jnp.full((512, 1024), -jnp.inf, F32)
    bank_refs = (bW1, bW2, bW3, bW4, bW5, bW6, bW7, bW8)
    for k in range(1, 9):
        Wk = bank_refs[k - 1]
        acc = jnp.zeros((L, 128), F32)
        for j in range(k):
            s = j - k // 2 + 8
            acc = acc + _dot(xp_scr[s:s + L], Wk[j])
        yk = jnp.maximum(acc * bscale_ref[k - 1] + bshift_ref[k - 1], 0.0)
        bank_scr[0:L, (k - 1) * 128:k * 128] = yk

    pooled = jnp.maximum(bank_scr[0:L], bank_scr[1:L + 1])

    pp_scr[:] = jnp.concatenate(
        [jnp.zeros((8, 1024), F32), pooled, jnp.zeros((4, 1024), F32)], axis=0)
    c1 = jnp.zeros((L, 256), F32)
    for j in range(3):
        c1 = c1 + _dot(pp_scr[7 + j:7 + j + L], p1W_ref[j])
    c1 = jnp.maximum(c1 * p1s_ref[:] + p1sh_ref[:], 0.0)

    c1p_scr[:] = jnp.concatenate(
        [jnp.zeros((8, 256), F32), c1, jnp.zeros((4, 256), F32)], axis=0)
    c2 = jnp.zeros((L, 128), F32)
    for j in range(3):
        c2 = c2 + _dot(c1p_scr[7 + j:7 + j + L], p2W_ref[j])
    xh = c2 * p2s_ref[:] + p2sh_ref[:] + x0

    for i in range(4):
        Hh = jnp.maximum(_dot(xh, hWh_ref[i]) + hbh_ref[i:i + 1], 0.0)
        Tt = jax.nn.sigmoid(_dot(xh, hWt_ref[i]) + hbt_ref[i:i + 1])
        xh = Hh * Tt + xh * (1.0 - Tt)
    out_ref[0] = xh


def _bigru_body(x_ref, Wih_ref, bih_ref, Whh_ref, bhh_ref, out_ref):
    Bsz = x_ref.shape[0]
    L = x_ref.shape[1] // 128
    UNROLL = 10
    pid = pl.program_id(0)
    Wih = Wih_ref[0]
    bih = bih_ref[0]
    Whh = Whh_ref[0]
    bhh = bhh_ref[0]

    def step(j, h):
        base = j * UNROLL
        idxs = [jnp.where(pid == 0, base + k, L - 1 - (base + k))
                for k in range(UNROLL)]
        gis = [_dot(x_ref[:, pl.ds(pl.multiple_of(idx * 128, 128), 128)],
                    Wih) + bih
               for idx in idxs]
        for k in range(UNROLL):
            gh = _dot(h, Whh) + bhh
            h = _gru_update(gis[k], gh, h)
            out_ref[0, pl.ds(idxs[k], 1)] = h.reshape(1, Bsz, 128)
        return h

    jax.lax.fori_loop(0, L // UNROLL, step, jnp.zeros((Bsz, 128), F32))


def _linear_body(xf_ref, xb_ref, Wf_ref, Wb_ref, b_ref, out_ref):
    out_ref[0] = (_dot(xf_ref[:], Wf_ref[:]) + _dot(xb_ref[:], Wb_ref[:])
                  + b_ref[:])


def _full_spec(shape):
    n = len(shape)
    return pl.BlockSpec(shape, lambda i, _n=n: (0,) * _n)


def kernel(z, y, lengths, params):
    p = params
    Bsz, Tlen, _ = z.shape
    T = y.shape[1] // R
    L = T * R
    Bc = Bsz // 2

    yr = y.reshape(Bsz, T, PRENET_IN)
    frames = jnp.concatenate(
        [jnp.zeros((Bsz, 1, PRENET_IN), z.dtype), yr[:, :-1]], axis=1)
    frames_t = jnp.swapaxes(frames, 0, 1)
    maskf = (jnp.arange(Tlen)[None, :] < lengths[:, None]).astype(F32)

    row = lambda b: b[None, :]

    arnn, g1, g2 = p['attn_rnn'], p['dec_gru1'], p['dec_gru2']
    dec_weights = (
        p['pre_W1'], row(p['pre_b1']), p['pre_W2'], row(p['pre_b2']),
        arnn['Wih'], row(arnn['bih']), arnn['Whh'], row(arnn['bhh']),
        p['Wq'], p['Wm'], row(p['v']).astype(jnp.bfloat16),
        p['proj_W'][:256], p['proj_W'][256:], row(p['proj_b']),
        g1['Wih'], row(g1['bih']), g1['Whh'], row(g1['bhh']),
        g2['Wih'], row(g2['bih']), g2['Whh'], row(g2['bhh']),
        p['mel_W'], row(p['mel_b']),
    )
    dec_in_specs = (
        [pl.BlockSpec((T, Bc, PRENET_IN), lambda i: (0, i, 0)),
         pl.BlockSpec((Bc, Tlen, 256), lambda i: (i, 0, 0)),
         pl.BlockSpec((Bc, Tlen), lambda i: (i, 0))]
        + [_full_spec(w.shape) for w in dec_weights])
    mels, aligns = pl.pallas_call(
        _decoder_body,
        grid=(2,),
        in_specs=dec_in_specs,
        out_specs=[pl.BlockSpec((T, Bc, PRENET_IN), lambda i: (0, i, 0)),
                   pl.BlockSpec((T, Bc, Tlen), lambda i: (0, i, 0))],
        out_shape=[jax.ShapeDtypeStruct((T, Bsz, PRENET_IN), F32),
                   jax.ShapeDtypeStruct((T, Bsz, Tlen), F32)],
        scratch_shapes=[pltpu.VMEM((T, Bc, 128), F32),
                        pltpu.VMEM((Bc, Tlen, 256), jnp.bfloat16),
                        pltpu.VMEM((T, Bc, 256), F32)],
        compiler_params=pltpu.CompilerParams(
            dimension_semantics=("parallel",),
            vmem_limit_bytes=56 * 1024 * 1024),
    )(frames_t, z, maskf, *dec_weights)

    mel_pred = jnp.swapaxes(mels, 0, 1).reshape(Bsz, L, N_MELS)
    alignments = jnp.swapaxes(aligns, 0, 1)

    def bn_scale_shift(bn):
        s = bn['gamma'] * jax.lax.rsqrt(bn['var'] + 1e-5)
        return s, bn['beta'] - bn['mean'] * s

    bss = [bn_scale_shift(bp['bn']) for bp in p['bank']]
    bscale = jnp.stack([s for s, _ in bss])
    bshift = jnp.stack([sh for _, sh in bss])
    p1s, p1sh = bn_scale_shift(p['proj1_bn'])
    p2s, p2sh = bn_scale_shift(p['proj2_bn'])
    hWh = jnp.stack([hp['Wh'] for hp in p['highway']])
    hbh = jnp.stack([hp['bh'] for hp in p['highway']])
    hWt = jnp.stack([hp['Wt'] for hp in p['highway']])
    hbt = jnp.stack([hp['bt'] for hp in p['highway']])
    cbhg_weights = (
        (p['pre_cbhg_W'],)
        + tuple(bp['W'] for bp in p['bank'])
        + (bscale, bshift,
           p['proj1_W'], row(p1s), row(p1sh),
           p['proj2_W'], row(p2s), row(p2sh),
           hWh, hbh, hWt, hbt))
    xcb = pl.pallas_call(
        _cbhg_body,
        grid=(Bsz,),
        in_specs=([pl.BlockSpec((1, L, N_MELS), lambda b: (b, 0, 0))]
                  + [_full_spec(w.shape) for w in cbhg_weights]),
        out_specs=pl.BlockSpec((1, L, 128), lambda b: (b, 0, 0)),
        out_shape=jax.ShapeDtypeStruct((Bsz, L, 128), F32),
        scratch_shapes=[pltpu.VMEM((512, 128), F32),
                        pltpu.VMEM((512, 1024), F32),
                        pltpu.VMEM((512, 1024), F32),
                        pltpu.VMEM((512, 256), F32)],
        compiler_params=pltpu.CompilerParams(
            dimension_semantics=("parallel",),
            vmem_limit_bytes=56 * 1024 * 1024),
    )(mel_pred, *cbhg_weights)

    xcb_flat = xcb.reshape(Bsz, L * 128)
    Wih_fb = jnp.stack([p['gru_f']['Wih'], p['gru_b']['Wih']])
    bih_fb = jnp.stack([row(p['gru_f']['bih']), row(p['gru_b']['bih'])])
    Whh_fb = jnp.stack([p['gru_f']['Whh'], p['gru_b']['Whh']])
    bhh_fb = jnp.stack([row(p['gru_f']['bhh']), row(p['gru_b']['bhh'])])
    h_all = pl.pallas_call(
        _bigru_body,
        grid=(2,),
        in_specs=[pl.BlockSpec((Bsz, L * 128), lambda i: (0, 0)),
                  pl.BlockSpec((1, 128, 384), lambda i: (i, 0, 0)),
                  pl.BlockSpec((1, 1, 384), lambda i: (i, 0, 0)),
                  pl.BlockSpec((1, 128, 384), lambda i: (i, 0, 0)),
                  pl.BlockSpec((1, 1, 384), lambda i: (i, 0, 0))],
        out_specs=pl.BlockSpec((1, L, Bsz, 128), lambda i: (i, 0, 0, 0)),
        out_shape=jax.ShapeDtypeStruct((2, L, Bsz, 128), F32),
        compiler_params=pltpu.CompilerParams(
            dimension_semantics=("parallel",),
            vmem_limit_bytes=56 * 1024 * 1024),
    )(xcb_flat, Wih_fb, bih_fb, Whh_fb, bhh_fb)

    xf = h_all[0].reshape(L, Bsz * 128)
    xb = h_all[1].reshape(L, Bsz * 128)
    lin_pred = pl.pallas_call(
        _linear_body,
        grid=(Bsz,),
        in_specs=[pl.BlockSpec((L, 128), lambda b: (0, b)),
                  pl.BlockSpec((L, 128), lambda b: (0, b)),
                  _full_spec((128, 1025)), _full_spec((128, 1025)),
                  _full_spec((1, 1025))],
        out_specs=pl.BlockSpec((1, L, 1025), lambda b: (b, 0, 0)),
        out_shape=jax.ShapeDtypeStruct((Bsz, L, 1025), F32),
        compiler_params=pltpu.CompilerParams(
            dimension_semantics=("parallel",),
            vmem_limit_bytes=56 * 1024 * 1024),
    )(xf, xb, p['lin_W'][:128], p['lin_W'][128:], row(p['lin_b']))

    return mel_pred, lin_pred, alignments


# speedup vs baseline: 1.0562x; 1.0239x over previous
import jax
import jax.numpy as jnp
from jax.experimental import pallas as pl
from jax.experimental.pallas import tpu as pltpu

N_MELS, R = 80, 5
PRENET_IN = N_MELS * R

F32 = jnp.float32


def _gru_update(gi, gh, h):
    H = h.shape[-1]
    r = jax.nn.sigmoid(gi[:, :H] + gh[:, :H])
    u = jax.nn.sigmoid(gi[:, H:2 * H] + gh[:, H:2 * H])
    n = jnp.tanh(gi[:, 2 * H:] + r * gh[:, 2 * H:])
    return (1.0 - u) * n + u * h


def _dot(a, b):
    return jnp.dot(a, b, preferred_element_type=F32)


def _decoder_body(frames_ref, z_ref, maskf_ref,
                  preW1_ref, preb1_ref, preW2_ref, preb2_ref,
                  aWih_ref, abih_ref, aWhh_ref, abhh_ref,
                  Wq_ref, Wm_ref, v_ref,
                  pW_c_ref, pW_h_ref, pb_ref,
                  g1Wih_ref, g1bih_ref, g1Whh_ref, g1bhh_ref,
                  g2Wih_ref, g2bih_ref, g2Whh_ref, g2bhh_ref,
                  melW_ref, melb_ref,
                  mel_ref, al_ref,
                  pren_scr, zm_scr, xs_scr):
    T, Bc, _ = frames_ref.shape
    Tlen = z_ref.shape[1]

    fr = frames_ref[:].reshape(T * Bc, PRENET_IN)
    p1 = jnp.maximum(_dot(fr, preW1_ref[:]) + preb1_ref[:], 0.0)
    p2 = jnp.maximum(_dot(p1, preW2_ref[:]) + preb2_ref[:], 0.0)
    pren_scr[:] = p2.reshape(T, Bc, 128)

    zf = z_ref[:].reshape(Bc * Tlen, 256)
    zm_scr[:] = _dot(zf, Wm_ref[:]).astype(jnp.bfloat16).reshape(
        Bc, Tlen, 256)

    maskf = maskf_ref[:]
    v = v_ref[:]

    def step(t, carry):
        ha, h1, h2, ctx = carry
        pt = pren_scr[pl.ds(t, 1)].reshape(Bc, 128)
        gi = _dot(jnp.concatenate([pt, ctx], -1), aWih_ref[:]) + abih_ref[:]
        gh = _dot(ha, aWhh_ref[:]) + abhh_ref[:]
        ha = _gru_update(gi, gh, ha)
        q = _dot(ha, Wq_ref[:]).astype(jnp.bfloat16)
        tt = jnp.tanh(zm_scr[:] + q[:, None, :])
        e = jnp.sum((tt * v[None, :, :].reshape(1, 1, 256)).astype(F32),
                    axis=-1)
        e = jnp.where(maskf > 0, e, -1e9)
        m = jnp.max(e, axis=-1, keepdims=True)
        ex = jnp.exp(e - m)
        a = ex / jnp.sum(ex, axis=-1, keepdims=True)
        ctx = jnp.sum(a[:, :, None] * z_ref[:], axis=1)
        x = _dot(ctx, pW_c_ref[:]) + _dot(ha, pW_h_ref[:]) + pb_ref[:]
        h1 = _gru_update(_dot(x, g1Wih_ref[:]) + g1bih_ref[:],
                         _dot(h1, g1Whh_ref[:]) + g1bhh_ref[:], h1)
        x = x + h1
        h2 = _gru_update(_dot(x, g2Wih_ref[:]) + g2bih_ref[:],
                         _dot(h2, g2Whh_ref[:]) + g2bhh_ref[:], h2)
        x = x + h2
        xs_scr[pl.ds(t, 1)] = x.reshape(1, Bc, 256)
        al_ref[pl.ds(t, 1)] = a.reshape(1, Bc, Tlen)
        return ha, h1, h2, ctx

    z0 = jnp.zeros((Bc, 256), F32)
    jax.lax.fori_loop(0, T, step, (z0, z0, z0, z0))

    xs = xs_scr[:].reshape(T * Bc, 256)
    mel_ref[:] = (_dot(xs, melW_ref[:]) + melb_ref[:]).reshape(
        T, Bc, PRENET_IN)


def _cbhg_body(mel_ref, preW_ref,
               bW1, bW2, bW3, bW4, bW5, bW6, bW7, bW8,
               bscale_ref, bshift_ref,
               p1W_ref, p1s_ref, p1sh_ref,
               p2W_ref, p2s_ref, p2sh_ref,
               hWh_ref, hbh_ref, hWt_ref, hbt_ref,
               out_ref,
               xp_scr, bank_scr, pp_scr, c1p_scr):
    L = mel_ref.shape[1]
    x = mel_ref[0]
    x0 = _dot(x, preW_ref[:])

    xp_scr[:] = jnp.concatenate(
        [jnp.zeros((8, 128), F32), x0, jnp.zeros((4, 128), F32)], axis=0)
    bank_scr[:] = jnp.full((512, 1024), -jnp.inf, F32)
    bank_refs = (bW1, bW2, bW3, bW4, bW5, bW6, bW7, bW8)
    for k in range(1, 9):
        Wk = bank_refs[k - 1]
        acc = jnp.zeros((L, 128), F32)
        for j in range(0, k - 1, 2):
            s = j - k // 2 + 8
            x2 = jnp.concatenate(
                [xp_scr[s:s + L], xp_scr[s + 1:s + 1 + L]], axis=-1)
            w2 = jnp.concatenate([Wk[j], Wk[j + 1]], axis=0)
            acc = acc + _dot(x2, w2)
        if k % 2:
            s = (k - 1) - k // 2 + 8
            acc = acc + _dot(xp_scr[s:s + L], Wk[k - 1])
        yk = jnp.maximum(acc * bscale_ref[k - 1] + bshift_ref[k - 1], 0.0)
        bank_scr[0:L, (k - 1) * 128:k * 128] = yk

    pooled = jnp.maximum(bank_scr[0:L], bank_scr[1:L + 1])

    pp_scr[:] = jnp.concatenate(
        [jnp.zeros((8, 1024), F32), pooled, jnp.zeros((4, 1024), F32)], axis=0)
    c1 = jnp.zeros((L, 256), F32)
    for j in range(3):
        c1 = c1 + _dot(pp_scr[7 + j:7 + j + L], p1W_ref[j])
    c1 = jnp.maximum(c1 * p1s_ref[:] + p1sh_ref[:], 0.0)

    c1p_scr[:] = jnp.concatenate(
        [jnp.zeros((8, 256), F32), c1, jnp.zeros((4, 256), F32)], axis=0)
    c2 = jnp.zeros((L, 128), F32)
    for j in range(3):
        c2 = c2 + _dot(c1p_scr[7 + j:7 + j + L], p2W_ref[j])
    xh = c2 * p2s_ref[:] + p2sh_ref[:] + x0

    for i in range(4):
        Hh = jnp.maximum(_dot(xh, hWh_ref[i]) + hbh_ref[i:i + 1], 0.0)
        Tt = jax.nn.sigmoid(_dot(xh, hWt_ref[i]) + hbt_ref[i:i + 1])
        xh = Hh * Tt + xh * (1.0 - Tt)
    out_ref[0] = xh


def _bigru_body(x_ref, Wih_ref, bih_ref, Whh_ref, bhh_ref, out_ref):
    Bsz = x_ref.shape[0]
    L = x_ref.shape[1] // 128
    UNROLL = 10
    pid = pl.program_id(0)
    Wih = Wih_ref[0]
    bih = bih_ref[0]
    Whh = Whh_ref[0]
    bhh = bhh_ref[0]

    def step(j, h):
        base = j * UNROLL
        idxs = [jnp.where(pid == 0, base + k, L - 1 - (base + k))
                for k in range(UNROLL)]
        gis = [_dot(x_ref[:, pl.ds(pl.multiple_of(idx * 128, 128), 128)],
                    Wih) + bih
               for idx in idxs]
        for k in range(UNROLL):
            gh = _dot(h, Whh) + bhh
            h = _gru_update(gis[k], gh, h)
            out_ref[0, pl.ds(idxs[k], 1)] = h.reshape(1, Bsz, 128)
        return h

    jax.lax.fori_loop(0, L // UNROLL, step, jnp.zeros((Bsz, 128), F32))


def _linear_body(xf_ref, xb_ref, Wf_ref, Wb_ref, b_ref, out_ref):
    out_ref[0] = (_dot(xf_ref[:], Wf_ref[:]) + _dot(xb_ref[:], Wb_ref[:])
                  + b_ref[:])


def _full_spec(shape):
    n = len(shape)
    return pl.BlockSpec(shape, lambda i, _n=n: (0,) * _n)


def kernel(z, y, lengths, params):
    p = params
    Bsz, Tlen, _ = z.shape
    T = y.shape[1] // R
    L = T * R
    Bc = Bsz // 2

    yr = y.reshape(Bsz, T, PRENET_IN)
    frames = jnp.concatenate(
        [jnp.zeros((Bsz, 1, PRENET_IN), z.dtype), yr[:, :-1]], axis=1)
    frames_t = jnp.swapaxes(frames, 0, 1)
    maskf = (jnp.arange(Tlen)[None, :] < lengths[:, None]).astype(F32)

    row = lambda b: b[None, :]

    arnn, g1, g2 = p['attn_rnn'], p['dec_gru1'], p['dec_gru2']
    dec_weights = (
        p['pre_W1'], row(p['pre_b1']), p['pre_W2'], row(p['pre_b2']),
        arnn['Wih'], row(arnn['bih']), arnn['Whh'], row(arnn['bhh']),
        p['Wq'], p['Wm'], row(p['v']).astype(jnp.bfloat16),
        p['proj_W'][:256], p['proj_W'][256:], row(p['proj_b']),
        g1['Wih'], row(g1['bih']), g1['Whh'], row(g1['bhh']),
        g2['Wih'], row(g2['bih']), g2['Whh'], row(g2['bhh']),
        p['mel_W'], row(p['mel_b']),
    )
    dec_in_specs = (
        [pl.BlockSpec((T, Bc, PRENET_IN), lambda i: (0, i, 0)),
         pl.BlockSpec((Bc, Tlen, 256), lambda i: (i, 0, 0)),
         pl.BlockSpec((Bc, Tlen), lambda i: (i, 0))]
        + [_full_spec(w.shape) for w in dec_weights])
    mels, aligns = pl.pallas_call(
        _decoder_body,
        grid=(2,),
        in_specs=dec_in_specs,
        out_specs=[pl.BlockSpec((T, Bc, PRENET_IN), lambda i: (0, i, 0)),
                   pl.BlockSpec((T, Bc, Tlen), lambda i: (0, i, 0))],
        out_shape=[jax.ShapeDtypeStruct((T, Bsz, PRENET_IN), F32),
                   jax.ShapeDtypeStruct((T, Bsz, Tlen), F32)],
        scratch_shapes=[pltpu.VMEM((T, Bc, 128), F32),
                        pltpu.VMEM((Bc, Tlen, 256), jnp.bfloat16),
                        pltpu.VMEM((T, Bc, 256), F32)],
        compiler_params=pltpu.CompilerParams(
            dimension_semantics=("parallel",),
            vmem_limit_bytes=56 * 1024 * 1024),
    )(frames_t, z, maskf, *dec_weights)

    mel_pred = jnp.swapaxes(mels, 0, 1).reshape(Bsz, L, N_MELS)
    alignments = jnp.swapaxes(aligns, 0, 1)

    def bn_scale_shift(bn):
        s = bn['gamma'] * jax.lax.rsqrt(bn['var'] + 1e-5)
        return s, bn['beta'] - bn['mean'] * s

    bss = [bn_scale_shift(bp['bn']) for bp in p['bank']]
    bscale = jnp.stack([s for s, _ in bss])
    bshift = jnp.stack([sh for _, sh in bss])
    p1s, p1sh = bn_scale_shift(p['proj1_bn'])
    p2s, p2sh = bn_scale_shift(p['proj2_bn'])
    hWh = jnp.stack([hp['Wh'] for hp in p['highway']])
    hbh = jnp.stack([hp['bh'] for hp in p['highway']])
    hWt = jnp.stack([hp['Wt'] for hp in p['highway']])
    hbt = jnp.stack([hp['bt'] for hp in p['highway']])
    cbhg_weights = (
        (p['pre_cbhg_W'],)
        + tuple(bp['W'] for bp in p['bank'])
        + (bscale, bshift,
           p['proj1_W'], row(p1s), row(p1sh),
           p['proj2_W'], row(p2s), row(p2sh),
           hWh, hbh, hWt, hbt))
    xcb = pl.pallas_call(
        _cbhg_body,
        grid=(Bsz,),
        in_specs=([pl.BlockSpec((1, L, N_MELS), lambda b: (b, 0, 0))]
                  + [_full_spec(w.shape) for w in cbhg_weights]),
        out_specs=pl.BlockSpec((1, L, 128), lambda b: (b, 0, 0)),
        out_shape=jax.ShapeDtypeStruct((Bsz, L, 128), F32),
        scratch_shapes=[pltpu.VMEM((512, 128), F32),
                        pltpu.VMEM((512, 1024), F32),
                        pltpu.VMEM((512, 1024), F32),
                        pltpu.VMEM((512, 256), F32)],
        compiler_params=pltpu.CompilerParams(
            dimension_semantics=("parallel",),
            vmem_limit_bytes=56 * 1024 * 1024),
    )(mel_pred, *cbhg_weights)

    xcb_flat = xcb.reshape(Bsz, L * 128)
    Wih_fb = jnp.stack([p['gru_f']['Wih'], p['gru_b']['Wih']])
    bih_fb = jnp.stack([row(p['gru_f']['bih']), row(p['gru_b']['bih'])])
    Whh_fb = jnp.stack([p['gru_f']['Whh'], p['gru_b']['Whh']])
    bhh_fb = jnp.stack([row(p['gru_f']['bhh']), row(p['gru_b']['bhh'])])
    h_all = pl.pallas_call(
        _bigru_body,
        grid=(2,),
        in_specs=[pl.BlockSpec((Bsz, L * 128), lambda i: (0, 0)),
                  pl.BlockSpec((1, 128, 384), lambda i: (i, 0, 0)),
                  pl.BlockSpec((1, 1, 384), lambda i: (i, 0, 0)),
                  pl.BlockSpec((1, 128, 384), lambda i: (i, 0, 0)),
                  pl.BlockSpec((1, 1, 384), lambda i: (i, 0, 0))],
        out_specs=pl.BlockSpec((1, L, Bsz, 128), lambda i: (i, 0, 0, 0)),
        out_shape=jax.ShapeDtypeStruct((2, L, Bsz, 128), F32),
        compiler_params=pltpu.CompilerParams(
            dimension_semantics=("parallel",),
            vmem_limit_bytes=56 * 1024 * 1024),
    )(xcb_flat, Wih_fb, bih_fb, Whh_fb, bhh_fb)

    xf = h_all[0].reshape(L, Bsz * 128)
    xb = h_all[1].reshape(L, Bsz * 128)
    lin_pred = pl.pallas_call(
        _linear_body,
        grid=(Bsz,),
        in_specs=[pl.BlockSpec((L, 128), lambda b: (0, b)),
                  pl.BlockSpec((L, 128), lambda b: (0, b)),
                  _full_spec((128, 1025)), _full_spec((128, 1025)),
                  _full_spec((1, 1025))],
        out_specs=pl.BlockSpec((1, L, 1025), lambda b: (b, 0, 0)),
        out_shape=jax.ShapeDtypeStruct((Bsz, L, 1025), F32),
        compiler_params=pltpu.CompilerParams(
            dimension_semantics=("parallel",),
            vmem_limit_bytes=56 * 1024 * 1024),
    )(xf, xb, p['lin_W'][:128], p['lin_W'][128:], row(p['lin_b']))

    return mel_pred, lin_pred, alignments
